# baseline (speedup 1.0000x reference)
import functools

import jax
import jax.numpy as jnp
from jax import lax
from jax.experimental import pallas as pl
from jax.experimental.pallas import tpu as pltpu
from jax.experimental.pallas import tpu_sc as plsc

V = 100000
D = 64
DP = 128
BATCH = 4096
HIST = 50
NC, NS = 2, 16
NW = NC * NS
L = 16

NCOL = V // DP
TAIL = V - NCOL * DP
CPW = (NCOL + NW - 1) // NW

PB = BATCH // NW
K = 5
NIT = HIST // K

_mesh = plsc.VectorSubcoreMesh(core_axis_name="c", subcore_axis_name="s")


def _worker_id():
    return lax.axis_index("s") * NC + lax.axis_index("c")


def _iotas():
    return [lax.iota(jnp.int32, L) + L * k for k in range(DP // L)]


@functools.partial(
    pl.kernel,
    mesh=_mesh,
    out_type=jax.ShapeDtypeStruct((V, DP), jnp.float32),
    compiler_params=pltpu.CompilerParams(use_tc_tiling_on_sc=True, needs_layout_passes=False),
    scratch_types=[
        pltpu.VMEM((2, D, DP), jnp.float32),
        pltpu.VMEM((2, DP, DP), jnp.float32),
        pltpu.VMEM((D, TAIL), jnp.float32),
        pltpu.VMEM((TAIL, DP), jnp.float32),
        pltpu.SemaphoreType.DMA((2,)),
        pltpu.SemaphoreType.DMA((2,)),
    ],
)
def _row_majorize(tab_t_hbm, out_hbm, in_v, tr_v, tin_v, ttr_v, isem, osem):
    wid = _worker_id()
    iotas = _iotas()

    def col0(t):
        return (wid + t * NW) * DP

    def start_in(t, s):
        pltpu.async_copy(
            tab_t_hbm.at[:, pl.ds(col0(t), DP)], in_v.at[s], isem.at[s]
        )

    def wait_in(t, s):
        pltpu.make_async_copy(
            tab_t_hbm.at[:, pl.ds(col0(t), DP)], in_v.at[s], isem.at[s]
        ).wait()

    def start_out(t, s):
        pltpu.async_copy(
            tr_v.at[s], out_hbm.at[pl.ds(col0(t), DP)], osem.at[s]
        )

    def drain_out(t, s):
        pltpu.make_async_copy(
            tr_v.at[s], out_hbm.at[pl.ds(col0(t), DP)], osem.at[s]
        ).wait()

    def transpose_slab(s):
        def rbody(r, carry):
            for k in range(D // L):
                vals = plsc.load_gather(
                    in_v.at[s], [iotas[k], jnp.full((L,), r, jnp.int32)]
                )
                tr_v[s, r, pl.ds(L * k, L)] = vals
            return carry

        lax.fori_loop(0, DP, rbody, 0)

    nsteps = jnp.where(wid + (CPW - 1) * NW < NCOL, CPW, CPW - 1)

    @pl.when(nsteps > 0)
    def _():
        start_in(0, 0)

    @pl.when(nsteps > 1)
    def _():
        start_in(1, 1)

    def body(it, carry):
        for u in (0, 1):
            t = 2 * it + u

            @pl.when(t < nsteps)
            def _():
                wait_in(t, u)
                pl.when(t >= 2)(lambda: drain_out(t - 2, u))
                transpose_slab(u)
                pl.when(t + 2 < nsteps)(lambda: start_in(t + 2, u))
                start_out(t, u)
        return carry

    lax.fori_loop(0, (CPW + 1) // 2, body, 0)
    for u in (0, 1):
        pl.when(nsteps > u)(
            lambda u=u: drain_out(jnp.maximum(nsteps - 2 + u, 0), u)
        )

    @pl.when(wid == NW - 1)
    def _():
        for d in range(D):
            pltpu.sync_copy(
                tab_t_hbm.at[d, pl.ds(NCOL * DP, TAIL)], tin_v.at[d]
            )

        def rbody(r, carry):
            for k in range(D // L):
                vals = plsc.load_gather(
                    tin_v, [iotas[k], jnp.full((L,), r, jnp.int32)]
                )
                ttr_v[r, pl.ds(L * k, L)] = vals
            return carry

        lax.fori_loop(0, TAIL, rbody, 0)
        pltpu.sync_copy(ttr_v, out_hbm.at[pl.ds(NCOL * DP, TAIL)])


@functools.partial(
    pl.kernel,
    mesh=_mesh,
    out_type=jax.ShapeDtypeStruct((HIST, D, BATCH), jnp.float32),
    compiler_params=pltpu.CompilerParams(use_tc_tiling_on_sc=True, needs_layout_passes=False),
    scratch_types=[
        pltpu.VMEM((HIST, PB), jnp.int32),
        pltpu.VMEM((K, PB, DP), jnp.float32),
        pltpu.VMEM((D, PB), jnp.float32),
        pltpu.SemaphoreType.DMA((K,)),
        pltpu.SemaphoreType.DMA,
    ],
)
def _gather_t(idx_t_hbm, tab_hbm, out_hbm, idx_v, gb_v, tb_v, gsem, wsem):
    wid = _worker_id()
    b0 = wid * PB
    iotas = _iotas()
    pltpu.sync_copy(idx_t_hbm.at[:, pl.ds(b0, PB)], idx_v)

    def gather(h, s):
        pltpu.async_copy(tab_hbm.at[idx_v.at[h]], gb_v.at[s], gsem.at[s])

    def wait_gather(h, s):
        pltpu.make_async_copy(
            tab_hbm.at[idx_v.at[h]], gb_v.at[s], gsem.at[s]
        ).wait()

    def start_write(h):
        pltpu.async_copy(tb_v, out_hbm.at[h, :, pl.ds(b0, PB)], wsem)

    def drain_write(h):
        pltpu.make_async_copy(
            tb_v, out_hbm.at[h, :, pl.ds(b0, PB)], wsem
        ).wait()

    def transpose_slab(s):
        def dbody(d, carry):
            didx = jnp.full((L,), d, jnp.int32)
            for k in range(PB // L):
                vals = plsc.load_gather(gb_v.at[s], [iotas[k], didx])
                tb_v[d, pl.ds(L * k, L)] = vals
            return carry

        lax.fori_loop(0, D, dbody, 0)

    for u in range(K):
        gather(u, u)

    def body(it, carry):
        h0 = it * K
        for u in range(K):
            h = h0 + u
            wait_gather(h, u)
            pl.when(h > 0)(lambda: drain_write(h - 1))
            transpose_slab(u)
            pl.when(it < NIT - 1)(lambda: gather(h + K, u))
            start_write(h)
        return carry

    lax.fori_loop(0, NIT, body, 0)
    drain_write(HIST - 1)


def kernel(inputs, table):
    tab_row_major = _row_majorize(table.T)
    out_t = _gather_t(inputs.T.astype(jnp.int32), tab_row_major)
    return out_t.transpose(2, 0, 1)


# speedup vs baseline: 2.2344x; 2.2344x over previous
import functools

import jax
import jax.numpy as jnp
from jax import lax
from jax.experimental import pallas as pl
from jax.experimental.pallas import tpu as pltpu
from jax.experimental.pallas import tpu_sc as plsc

V = 100000
D = 64
DP = 128
BATCH = 4096
HIST = 50
NC, NS = 2, 16
NW = NC * NS
L = 16

NCOL = V // DP
TAIL = V - NCOL * DP
CPW = (NCOL + NW - 1) // NW

PB = BATCH // NW
K = 5
NIT = HIST // K

_mesh = plsc.VectorSubcoreMesh(core_axis_name="c", subcore_axis_name="s")


def _worker_id():
    return lax.axis_index("s") * NC + lax.axis_index("c")


def _iotas():
    return [lax.iota(jnp.int32, L) + L * k for k in range(DP // L)]


@functools.partial(
    pl.kernel,
    mesh=_mesh,
    out_type=jax.ShapeDtypeStruct((V, DP), jnp.float32),
    compiler_params=pltpu.CompilerParams(use_tc_tiling_on_sc=True, needs_layout_passes=False),
    scratch_types=[
        pltpu.VMEM((2, D, DP), jnp.float32),
        pltpu.VMEM((2, DP, DP), jnp.float32),
        pltpu.VMEM((D, TAIL), jnp.float32),
        pltpu.VMEM((TAIL, DP), jnp.float32),
        pltpu.SemaphoreType.DMA((2,)),
        pltpu.SemaphoreType.DMA((2,)),
    ],
)
def _row_majorize(tab_t_hbm, out_hbm, in_v, tr_v, tin_v, ttr_v, isem, osem):
    wid = _worker_id()
    iotas = _iotas()

    def col0(t):
        return (wid + t * NW) * DP

    def start_in(t, s):
        pltpu.async_copy(
            tab_t_hbm.at[:, pl.ds(col0(t), DP)], in_v.at[s], isem.at[s]
        )

    def wait_in(t, s):
        pltpu.make_async_copy(
            tab_t_hbm.at[:, pl.ds(col0(t), DP)], in_v.at[s], isem.at[s]
        ).wait()

    def start_out(t, s):
        pltpu.async_copy(
            tr_v.at[s], out_hbm.at[pl.ds(col0(t), DP)], osem.at[s]
        )

    def drain_out(t, s):
        pltpu.make_async_copy(
            tr_v.at[s], out_hbm.at[pl.ds(col0(t), DP)], osem.at[s]
        ).wait()

    lane = lax.iota(jnp.int32, L)

    def transpose_slab(s):
        def rbody(r0, carry):
            rows = (lane + r0) & (DP - 1)
            for k in range(D // L):
                vals = plsc.load_gather(in_v.at[s], [iotas[k], rows])
                plsc.store_scatter(tr_v.at[s], [rows, iotas[k]], vals)
            return carry

        lax.fori_loop(0, DP, rbody, 0)

    nsteps = jnp.where(wid + (CPW - 1) * NW < NCOL, CPW, CPW - 1)

    @pl.when(nsteps > 0)
    def _():
        start_in(0, 0)

    @pl.when(nsteps > 1)
    def _():
        start_in(1, 1)

    def body(it, carry):
        for u in (0, 1):
            t = 2 * it + u

            @pl.when(t < nsteps)
            def _():
                wait_in(t, u)
                pl.when(t >= 2)(lambda: drain_out(t - 2, u))
                transpose_slab(u)
                pl.when(t + 2 < nsteps)(lambda: start_in(t + 2, u))
                start_out(t, u)
        return carry

    lax.fori_loop(0, (CPW + 1) // 2, body, 0)
    for u in (0, 1):
        pl.when(nsteps > u)(
            lambda u=u: drain_out(jnp.maximum(nsteps - 2 + u, 0), u)
        )

    @pl.when(wid == NW - 1)
    def _():
        for d in range(D):
            pltpu.sync_copy(
                tab_t_hbm.at[d, pl.ds(NCOL * DP, TAIL)], tin_v.at[d]
            )

        def rbody(r, carry):
            for k in range(D // L):
                vals = plsc.load_gather(
                    tin_v, [iotas[k], jnp.full((L,), r, jnp.int32)]
                )
                ttr_v[r, pl.ds(L * k, L)] = vals
            return carry

        lax.fori_loop(0, TAIL, rbody, 0)
        pltpu.sync_copy(ttr_v, out_hbm.at[pl.ds(NCOL * DP, TAIL)])


@functools.partial(
    pl.kernel,
    mesh=_mesh,
    out_type=jax.ShapeDtypeStruct((HIST, D, BATCH), jnp.float32),
    compiler_params=pltpu.CompilerParams(use_tc_tiling_on_sc=True, needs_layout_passes=False),
    scratch_types=[
        pltpu.VMEM((HIST, PB), jnp.int32),
        pltpu.VMEM((K, PB, DP), jnp.float32),
        pltpu.VMEM((D, PB), jnp.float32),
        pltpu.SemaphoreType.DMA((K,)),
        pltpu.SemaphoreType.DMA,
    ],
)
def _gather_t(idx_t_hbm, tab_hbm, out_hbm, idx_v, gb_v, tb_v, gsem, wsem):
    wid = _worker_id()
    b0 = wid * PB
    iotas = _iotas()
    pltpu.sync_copy(idx_t_hbm.at[:, pl.ds(b0, PB)], idx_v)

    def gather(h, s):
        pltpu.async_copy(tab_hbm.at[idx_v.at[h]], gb_v.at[s], gsem.at[s])

    def wait_gather(h, s):
        pltpu.make_async_copy(
            tab_hbm.at[idx_v.at[h]], gb_v.at[s], gsem.at[s]
        ).wait()

    def start_write(h):
        pltpu.async_copy(tb_v, out_hbm.at[h, :, pl.ds(b0, PB)], wsem)

    def drain_write(h):
        pltpu.make_async_copy(
            tb_v, out_hbm.at[h, :, pl.ds(b0, PB)], wsem
        ).wait()

    lane = lax.iota(jnp.int32, L)

    def transpose_slab(s):
        def bbody(b0, carry):
            rows = (lane + b0) & (PB - 1)
            for k in range(D // L):
                vals = plsc.load_gather(gb_v.at[s], [rows, iotas[k]])
                plsc.store_scatter(tb_v, [iotas[k], rows], vals)
            return carry

        lax.fori_loop(0, PB, bbody, 0)

    for u in range(K):
        gather(u, u)

    def body(it, carry):
        h0 = it * K
        for u in range(K):
            h = h0 + u
            wait_gather(h, u)
            pl.when(h > 0)(lambda: drain_write(h - 1))
            transpose_slab(u)
            pl.when(it < NIT - 1)(lambda: gather(h + K, u))
            start_write(h)
        return carry

    lax.fori_loop(0, NIT, body, 0)
    drain_write(HIST - 1)


def kernel(inputs, table):
    tab_row_major = _row_majorize(table.T)
    out_t = _gather_t(inputs.T.astype(jnp.int32), tab_row_major)
    return out_t.transpose(2, 0, 1)


# speedup vs baseline: 2.7334x; 1.2233x over previous
import functools

import jax
import jax.numpy as jnp
from jax import lax
from jax.experimental import pallas as pl
from jax.experimental.pallas import tpu as pltpu
from jax.experimental.pallas import tpu_sc as plsc

V = 100000
D = 64
DP = 128
BATCH = 4096
HIST = 50
NC, NS = 2, 16
NW = NC * NS
L = 16

NCOL = V // DP
TAIL = V - NCOL * DP
CPW = (NCOL + NW - 1) // NW

PB = BATCH // NW
K = 5
NIT = HIST // K

_mesh = plsc.VectorSubcoreMesh(core_axis_name="c", subcore_axis_name="s")


def _worker_id():
    return lax.axis_index("s") * NC + lax.axis_index("c")


def _iotas():
    return [lax.iota(jnp.int32, L) + L * k for k in range(DP // L)]


@functools.partial(
    pl.kernel,
    mesh=_mesh,
    out_type=jax.ShapeDtypeStruct((V, DP), jnp.float32),
    compiler_params=pltpu.CompilerParams(use_tc_tiling_on_sc=True, needs_layout_passes=False),
    scratch_types=[
        pltpu.VMEM((2, D, DP), jnp.float32),
        pltpu.VMEM((2, DP, DP), jnp.float32),
        pltpu.VMEM((D, TAIL), jnp.float32),
        pltpu.VMEM((TAIL, DP), jnp.float32),
        pltpu.SemaphoreType.DMA((2,)),
        pltpu.SemaphoreType.DMA((2,)),
    ],
)
def _row_majorize(tab_t_hbm, out_hbm, in_v, tr_v, tin_v, ttr_v, isem, osem):
    wid = _worker_id()
    iotas = _iotas()

    def col0(t):
        return (wid + t * NW) * DP

    def start_in(t, s):
        pltpu.async_copy(
            tab_t_hbm.at[:, pl.ds(col0(t), DP)], in_v.at[s], isem.at[s]
        )

    def wait_in(t, s):
        pltpu.make_async_copy(
            tab_t_hbm.at[:, pl.ds(col0(t), DP)], in_v.at[s], isem.at[s]
        ).wait()

    def start_out(t, s):
        pltpu.async_copy(
            tr_v.at[s], out_hbm.at[pl.ds(col0(t), DP)], osem.at[s]
        )

    def drain_out(t, s):
        pltpu.make_async_copy(
            tr_v.at[s], out_hbm.at[pl.ds(col0(t), DP)], osem.at[s]
        ).wait()

    lane = lax.iota(jnp.int32, L)

    def transpose_slab(s):
        def rbody(r0, carry):
            rows = (lane + r0) & (DP - 1)
            for k in range(D // L):
                vals = plsc.load_gather(in_v.at[s], [iotas[k], rows])
                plsc.store_scatter(tr_v.at[s], [rows, iotas[k]], vals)
            return carry

        lax.fori_loop(0, DP, rbody, 0)

    nsteps = jnp.where(wid + (CPW - 1) * NW < NCOL, CPW, CPW - 1)

    @pl.when(nsteps > 0)
    def _():
        start_in(0, 0)

    @pl.when(nsteps > 1)
    def _():
        start_in(1, 1)

    def body(it, carry):
        for u in (0, 1):
            t = 2 * it + u

            @pl.when(t < nsteps)
            def _():
                wait_in(t, u)
                pl.when(t >= 2)(lambda: drain_out(t - 2, u))
                transpose_slab(u)
                pl.when(t + 2 < nsteps)(lambda: start_in(t + 2, u))
                start_out(t, u)
        return carry

    lax.fori_loop(0, (CPW + 1) // 2, body, 0)
    for u in (0, 1):
        pl.when(nsteps > u)(
            lambda u=u: drain_out(jnp.maximum(nsteps - 2 + u, 0), u)
        )

    @pl.when(wid == NW - 1)
    def _():
        for d in range(D):
            pltpu.async_copy(
                tab_t_hbm.at[d, pl.ds(NCOL * DP, TAIL)], tin_v.at[d], isem.at[0]
            )
        for d in range(D):
            pltpu.make_async_copy(
                tab_t_hbm.at[d, pl.ds(NCOL * DP, TAIL)], tin_v.at[d], isem.at[0]
            ).wait()

        def rbody(r, carry):
            for k in range(D // L):
                vals = plsc.load_gather(
                    tin_v, [iotas[k], jnp.full((L,), r, jnp.int32)]
                )
                ttr_v[r, pl.ds(L * k, L)] = vals
            return carry

        lax.fori_loop(0, TAIL, rbody, 0)
        pltpu.sync_copy(ttr_v, out_hbm.at[pl.ds(NCOL * DP, TAIL)])


@functools.partial(
    pl.kernel,
    mesh=_mesh,
    out_type=jax.ShapeDtypeStruct((HIST, D, BATCH), jnp.float32),
    compiler_params=pltpu.CompilerParams(use_tc_tiling_on_sc=True, needs_layout_passes=False),
    scratch_types=[
        pltpu.VMEM((HIST, PB), jnp.int32),
        pltpu.VMEM((K, PB, DP), jnp.float32),
        pltpu.VMEM((K, D, PB), jnp.float32),
        pltpu.SemaphoreType.DMA((K,)),
        pltpu.SemaphoreType.DMA((K,)),
    ],
)
def _gather_t(idx_t_hbm, tab_hbm, out_hbm, idx_v, gb_v, tb_v, gsem, wsem):
    wid = _worker_id()
    b0 = wid * PB
    iotas = _iotas()
    pltpu.sync_copy(idx_t_hbm.at[:, pl.ds(b0, PB)], idx_v)

    def gather(h, s):
        pltpu.async_copy(tab_hbm.at[idx_v.at[h]], gb_v.at[s], gsem.at[s])

    def wait_gather(h, s):
        pltpu.make_async_copy(
            tab_hbm.at[idx_v.at[h]], gb_v.at[s], gsem.at[s]
        ).wait()

    def start_write(h, s):
        pltpu.async_copy(tb_v.at[s], out_hbm.at[h, :, pl.ds(b0, PB)], wsem.at[s])

    def drain_write(h, s):
        pltpu.make_async_copy(
            tb_v.at[s], out_hbm.at[h, :, pl.ds(b0, PB)], wsem.at[s]
        ).wait()

    lane = lax.iota(jnp.int32, L)

    def transpose_slab(s):
        def bbody(bb, carry):
            rows = (lane + bb) & (PB - 1)
            for k in range(D // L):
                vals = plsc.load_gather(gb_v.at[s], [rows, iotas[k]])
                plsc.store_scatter(tb_v.at[s], [iotas[k], rows], vals)
            return carry

        lax.fori_loop(0, PB, bbody, 0)

    for u in range(K):
        gather(u, u)

    def body(it, carry):
        h0 = it * K
        for u in range(K):
            h = h0 + u
            wait_gather(h, u)
            pl.when(it > 0)(lambda: drain_write(h - K, u))
            transpose_slab(u)
            pl.when(it < NIT - 1)(lambda: gather(h + K, u))
            start_write(h, u)
        return carry

    lax.fori_loop(0, NIT, body, 0)
    for u in range(K):
        drain_write(HIST - K + u, u)


def kernel(inputs, table):
    tab_row_major = _row_majorize(table.T)
    out_t = _gather_t(inputs.T.astype(jnp.int32), tab_row_major)
    return out_t.transpose(2, 0, 1)
